# baseline (speedup 1.0000x reference)
import functools

import jax
import jax.numpy as jnp
from jax import lax
from jax.experimental import pallas as pl
from jax.experimental.pallas import tpu as pltpu
from jax.experimental.pallas import tpu_sc as plsc

_LANES = 16
_CHUNK = 32


def _retile_body(in_ref, out_ref):
    x = in_ref[:, 0, :, :]
    xt = jnp.transpose(x, (1, 2, 0))
    bb = x.shape[0]
    xr = xt.reshape(1024, bb)
    out_ref[0, :, :] = xr[:out_ref.shape[1], :]


def _retile(x4, v):
    bb, tt = x4.shape[0], x4.shape[1]
    blk = 256
    return pl.pallas_call(
        _retile_body,
        grid=(tt, bb // blk),
        in_specs=[pl.BlockSpec((blk, 1, 8, 128), lambda t, j: (j, t, 0, 0))],
        out_specs=pl.BlockSpec((1, v, blk), lambda t, j: (t, 0, j)),
        out_shape=jax.ShapeDtypeStruct((tt, v, bb), jnp.float32),
    )(x4)


def _lse_body(table_ref, lse_ref):
    t = table_ref[...]
    m = jnp.max(t, axis=1, keepdims=True)
    s = jnp.sum(jnp.exp(t - m), axis=1, keepdims=True)
    lse_ref[...] = jnp.log(s) + m


def _row_lse(table):
    v = table.shape[0]
    return pl.pallas_call(
        _lse_body,
        out_shape=jax.ShapeDtypeStruct((v, 1), jnp.float32),
    )(table)


@functools.lru_cache(maxsize=None)
def _make_sc_kernel(bb, tt, v, d, nc, ns):
    nw = nc * ns
    nt = bb * tt
    per_w = nt // nw
    assert per_w * nw == nt
    n_chunks = per_w // _CHUNK
    assert n_chunks * _CHUNK == per_w and n_chunks % 2 == 0

    mesh = plsc.VectorSubcoreMesh(core_axis_name="c", subcore_axis_name="s")

    @functools.partial(
        pl.kernel,
        mesh=mesh,
        compiler_params=pltpu.CompilerParams(use_tc_tiling_on_sc=False),
        out_type=[
            jax.ShapeDtypeStruct((nt * 1024,), jnp.float32),
            jax.ShapeDtypeStruct((nw * _LANES,), jnp.float32),
        ],
        scratch_types=[
            pltpu.VMEM((per_w,), jnp.int32),
            pltpu.VMEM((per_w,), jnp.int32),
            pltpu.VMEM((per_w,), jnp.float32),
            pltpu.VMEM((per_w,), jnp.float32),
            pltpu.VMEM((_CHUNK, d), jnp.float32),
            pltpu.VMEM((_CHUNK, d), jnp.float32),
            pltpu.VMEM((_LANES,), jnp.float32),
            pltpu.SemaphoreType.DMA,
            pltpu.SemaphoreType.DMA,
            pltpu.SemaphoreType.DMA,
            pltpu.SemaphoreType.DMA,
            pltpu.SemaphoreType.DMA,
        ],
    )
    def sc_kernel(table_hbm, cat_hbm, idx_hbm, fidx_hbm,
                  out_hbm, part_hbm,
                  idx_v, fidx_v, picked_v, lsetok_v, buf0, buf1, acc_v,
                  gsem0, gsem1, osem0, osem1, psem):
        wid = lax.axis_index("s") * nc + lax.axis_index("c")
        base = wid * per_w
        pltpu.sync_copy(idx_hbm.at[pl.ds(base, per_w)], idx_v)
        pltpu.sync_copy(fidx_hbm.at[pl.ds(base, per_w)], fidx_v)
        pick_dma = pltpu.make_async_copy(
            cat_hbm.at[fidx_v], picked_v, psem)
        pick_dma.start()
        lse_dma = pltpu.make_async_copy(
            cat_hbm.at[idx_v], lsetok_v, psem)
        lse_dma.start()

        bufs = (buf0, buf1)
        gsems = (gsem0, gsem1)
        osems = (osem0, osem1)

        def gather(g, b):
            idx_slice = idx_v.at[pl.ds(g * _CHUNK, _CHUNK)]
            return pltpu.make_async_copy(
                table_hbm.at[idx_slice], bufs[b], gsems[b])

        def outcopy_start(g, b):
            off = (base + g * _CHUNK) * 1024
            for i in range(_CHUNK):
                pltpu.make_async_copy(
                    bufs[b].at[i], out_hbm.at[pl.ds(off + i * 1024, d)],
                    osems[b]).start()

        def outcopy_wait(g, b):
            off = (base + g * _CHUNK) * 1024
            for i in range(_CHUNK):
                pltpu.make_async_copy(
                    bufs[b].at[i], out_hbm.at[pl.ds(off + i * 1024, d)],
                    osems[b]).wait()

        gather(0, 0).start()
        gather(1, 1).start()

        def pair_body(p, carry):
            for b in range(2):
                g = 2 * p + b
                gather(g, b).wait()
                outcopy_start(g, b)

                @pl.when(g + 2 < n_chunks)
                def _():
                    outcopy_wait(g, b)
                    gather(g + 2, b).start()
            return carry

        lax.fori_loop(0, n_chunks // 2, pair_body, 0)
        outcopy_wait(n_chunks - 2, 0)
        outcopy_wait(n_chunks - 1, 1)

        pick_dma.wait()
        lse_dma.wait()
        acc_v[...] = jnp.zeros((_LANES,), jnp.float32)

        def loss_body(i, carry):
            o = i * _LANES
            acc_v[...] = acc_v[...] + (
                lsetok_v[pl.ds(o, _LANES)] - picked_v[pl.ds(o, _LANES)])
            return carry

        lax.fori_loop(0, per_w // _LANES, loss_body, 0)
        pltpu.sync_copy(acc_v, part_hbm.at[pl.ds(wid * _LANES, _LANES)])

    return sc_kernel


def kernel(idx, targets, table):
    b, t = idx.shape
    v, d = table.shape
    nt = b * t
    lse = _row_lse(table).reshape(v)
    idx_f = idx.reshape(nt).astype(jnp.int32)
    fidx = v + idx_f * d + targets.reshape(nt).astype(jnp.int32)
    cat = jnp.concatenate([lse, table.reshape(v * d)])
    info = plsc.get_sparse_core_info()
    sck = _make_sc_kernel(b, t, v, d, info.num_cores, info.num_subcores)
    logits_pad, parts = sck(table, cat, idx_f, fidx)
    loss = jnp.sum(parts) / nt
    x4 = logits_pad.reshape(b, t, 8, 128)
    out_t = _retile(x4, v)
    return jnp.transpose(out_t, (2, 0, 1)), loss


# speedup vs baseline: 3.9822x; 3.9822x over previous
import functools

import jax
import jax.numpy as jnp
from jax import lax
from jax.experimental import pallas as pl
from jax.experimental.pallas import tpu as pltpu
from jax.experimental.pallas import tpu_sc as plsc

_LANES = 16
_CHUNK = 32


def _retile_body(in_ref, out_ref):
    x = in_ref[0, 0, :, :]
    out_ref[0, :, :] = jnp.transpose(x)


def _retile(x4, v):
    npc, tt, bb = x4.shape[0], x4.shape[1], x4.shape[2]
    return pl.pallas_call(
        _retile_body,
        grid=(tt, npc),
        in_specs=[pl.BlockSpec((1, 1, bb, 128),
                               lambda t, c2: (c2, t, 0, 0))],
        out_specs=pl.BlockSpec((1, 128, bb), lambda t, c2: (t, c2, 0)),
        out_shape=jax.ShapeDtypeStruct((tt, v, bb), jnp.float32),
    )(x4)


def _lse_body(table_ref, lse_ref):
    t = table_ref[...]
    m = jnp.max(t, axis=1, keepdims=True)
    s = jnp.sum(jnp.exp(t - m), axis=1, keepdims=True)
    lse_ref[...] = jnp.log(s) + m


def _row_lse(table):
    v = table.shape[0]
    return pl.pallas_call(
        _lse_body,
        out_shape=jax.ShapeDtypeStruct((v, 1), jnp.float32),
    )(table)


@functools.lru_cache(maxsize=None)
def _make_sc_kernel(nt, v, d, dp, nc, ns):
    nw = nc * ns
    per_w = nt // nw
    assert per_w * nw == nt
    n_chunks = per_w // _CHUNK
    assert n_chunks * _CHUNK == per_w and n_chunks % 2 == 0
    npc = dp // 128

    mesh = plsc.VectorSubcoreMesh(core_axis_name="c", subcore_axis_name="s")

    @functools.partial(
        pl.kernel,
        mesh=mesh,
        compiler_params=pltpu.CompilerParams(use_tc_tiling_on_sc=False),
        out_type=[
            jax.ShapeDtypeStruct((npc * nt, 128), jnp.float32),
            jax.ShapeDtypeStruct((nw * _LANES,), jnp.float32),
        ],
        scratch_types=[
            pltpu.VMEM((per_w,), jnp.int32),
            pltpu.VMEM((per_w,), jnp.int32),
            pltpu.VMEM((per_w,), jnp.float32),
            pltpu.VMEM((per_w,), jnp.float32),
            pltpu.VMEM((_CHUNK, dp), jnp.float32),
            pltpu.VMEM((_CHUNK, dp), jnp.float32),
            pltpu.VMEM((_LANES,), jnp.float32),
            pltpu.SemaphoreType.DMA,
            pltpu.SemaphoreType.DMA,
            pltpu.SemaphoreType.DMA,
            pltpu.SemaphoreType.DMA,
            pltpu.SemaphoreType.DMA,
        ],
    )
    def sc_kernel(table_hbm, cat_hbm, idx_hbm, fidx_hbm,
                  out_hbm, part_hbm,
                  idx_v, fidx_v, picked_v, lsetok_v, buf0, buf1, acc_v,
                  gsem0, gsem1, osem0, osem1, psem):
        wid = lax.axis_index("s") * nc + lax.axis_index("c")
        base = wid * per_w
        pltpu.sync_copy(idx_hbm.at[pl.ds(base, per_w)], idx_v)
        pltpu.sync_copy(fidx_hbm.at[pl.ds(base, per_w)], fidx_v)
        pick_dma = pltpu.make_async_copy(
            cat_hbm.at[fidx_v], picked_v, psem)
        pick_dma.start()
        lse_dma = pltpu.make_async_copy(
            cat_hbm.at[idx_v], lsetok_v, psem)
        lse_dma.start()

        bufs = (buf0, buf1)
        gsems = (gsem0, gsem1)
        osems = (osem0, osem1)

        def gather(g, b):
            idx_slice = idx_v.at[pl.ds(g * _CHUNK, _CHUNK)]
            return pltpu.make_async_copy(
                table_hbm.at[idx_slice], bufs[b], gsems[b])

        def piece_copies(g, b):
            for c2 in range(npc):
                yield pltpu.make_async_copy(
                    bufs[b].at[:, pl.ds(c2 * 128, 128)],
                    out_hbm.at[pl.ds(c2 * nt + base + g * _CHUNK, _CHUNK)],
                    osems[b])

        def outcopy_start(g, b):
            for cp in piece_copies(g, b):
                cp.start()

        def outcopy_wait(g, b):
            for cp in piece_copies(g, b):
                cp.wait()

        gather(0, 0).start()
        gather(1, 1).start()

        def pair_body(p, carry):
            for b in range(2):
                g = 2 * p + b
                gather(g, b).wait()
                outcopy_start(g, b)

                @pl.when(g + 2 < n_chunks)
                def _():
                    outcopy_wait(g, b)
                    gather(g + 2, b).start()
            return carry

        lax.fori_loop(0, n_chunks // 2, pair_body, 0)
        outcopy_wait(n_chunks - 2, 0)
        outcopy_wait(n_chunks - 1, 1)

        pick_dma.wait()
        lse_dma.wait()
        acc_v[...] = jnp.zeros((_LANES,), jnp.float32)

        def loss_body(i, carry):
            o = i * _LANES
            acc_v[...] = acc_v[...] + (
                lsetok_v[pl.ds(o, _LANES)] - picked_v[pl.ds(o, _LANES)])
            return carry

        lax.fori_loop(0, per_w // _LANES, loss_body, 0)
        pltpu.sync_copy(acc_v, part_hbm.at[pl.ds(wid * _LANES, _LANES)])

    return sc_kernel


def kernel(idx, targets, table):
    b, t = idx.shape
    v, d = table.shape
    nt = b * t
    dp = 1024
    lse = _row_lse(table).reshape(v)
    idx_f = jnp.transpose(idx).reshape(nt).astype(jnp.int32)
    fidx = v + idx_f * d + jnp.transpose(targets).reshape(nt).astype(
        jnp.int32)
    cat = jnp.concatenate([lse, table.reshape(v * d)])
    table_p = jnp.pad(table, ((0, 0), (0, dp - d)))
    info = plsc.get_sparse_core_info()
    sck = _make_sc_kernel(nt, v, d, dp, info.num_cores, info.num_subcores)
    pieces, parts = sck(table_p, cat, idx_f, fidx)
    loss = jnp.sum(parts) / nt
    x4 = pieces.reshape(dp // 128, t, b, 128)
    out_t = _retile(x4, v)
    return jnp.transpose(out_t, (2, 0, 1)), loss


# speedup vs baseline: 3.9822x; 1.0000x over previous
import functools

import jax
import jax.numpy as jnp
from jax import lax
from jax.experimental import pallas as pl
from jax.experimental.pallas import tpu as pltpu
from jax.experimental.pallas import tpu_sc as plsc

_LANES = 16
_CHUNK = 32


def _retile_body(in_ref, out_ref):
    x = in_ref[0, 0, :, :]
    out_ref[0, :, :] = jnp.transpose(x)


def _retile(x4, v):
    npc, tt, bb = x4.shape[0], x4.shape[1], x4.shape[2]
    return pl.pallas_call(
        _retile_body,
        grid=(tt, npc),
        in_specs=[pl.BlockSpec((1, 1, bb, 128),
                               lambda t, c2: (c2, t, 0, 0))],
        out_specs=pl.BlockSpec((1, 128, bb), lambda t, c2: (t, c2, 0)),
        out_shape=jax.ShapeDtypeStruct((tt, v, bb), jnp.float32),
    )(x4)


def _lse_body(table_ref, lse_ref):
    t = table_ref[...]
    m = jnp.max(t, axis=1, keepdims=True)
    s = jnp.sum(jnp.exp(t - m), axis=1, keepdims=True)
    lse_ref[...] = jnp.log(s) + m


def _row_lse(table):
    v = table.shape[0]
    return pl.pallas_call(
        _lse_body,
        out_shape=jax.ShapeDtypeStruct((v, 1), jnp.float32),
    )(table)


@functools.lru_cache(maxsize=None)
def _make_sc_kernel(nt, v, d, dp, nc, ns):
    nw = nc * ns
    per_w = nt // nw
    assert per_w * nw == nt
    n_chunks = per_w // _CHUNK
    assert n_chunks * _CHUNK == per_w and n_chunks % 2 == 0
    npc = dp // 128

    mesh = plsc.VectorSubcoreMesh(core_axis_name="c", subcore_axis_name="s")

    @functools.partial(
        pl.kernel,
        mesh=mesh,
        compiler_params=pltpu.CompilerParams(use_tc_tiling_on_sc=False),
        out_type=[
            jax.ShapeDtypeStruct((npc * nt, 128), jnp.float32),
            jax.ShapeDtypeStruct((nw * _LANES,), jnp.float32),
        ],
        scratch_types=[
            pltpu.VMEM((per_w,), jnp.int32),
            pltpu.VMEM((per_w,), jnp.int32),
            pltpu.VMEM((per_w,), jnp.float32),
            pltpu.VMEM((per_w,), jnp.float32),
            pltpu.VMEM((_CHUNK, dp), jnp.float32),
            pltpu.VMEM((_CHUNK, dp), jnp.float32),
            pltpu.VMEM((_LANES,), jnp.float32),
            pltpu.SemaphoreType.DMA,
            pltpu.SemaphoreType.DMA,
            pltpu.SemaphoreType.DMA,
            pltpu.SemaphoreType.DMA,
            pltpu.SemaphoreType.DMA,
        ],
    )
    def sc_kernel(table_hbm, cat_hbm, idx_hbm, fidx_hbm,
                  out_hbm, part_hbm,
                  idx_v, fidx_v, picked_v, lsetok_v, buf0, buf1, acc_v,
                  gsem0, gsem1, osem0, osem1, psem):
        wid = lax.axis_index("s") * nc + lax.axis_index("c")
        base = wid * per_w
        pltpu.sync_copy(idx_hbm.at[pl.ds(base, per_w)], idx_v)
        pltpu.sync_copy(fidx_hbm.at[pl.ds(base, per_w)], fidx_v)

        def elem_gathers():
            for k in range(per_w // 128):
                s = pl.ds(k * 128, 128)
                yield pltpu.make_async_copy(
                    cat_hbm.at[fidx_v.at[s]], picked_v.at[s], psem)
                yield pltpu.make_async_copy(
                    cat_hbm.at[idx_v.at[s]], lsetok_v.at[s], psem)

        for eg in elem_gathers():
            eg.start()

        bufs = (buf0, buf1)
        gsems = (gsem0, gsem1)
        osems = (osem0, osem1)

        def gather(g, b):
            idx_slice = idx_v.at[pl.ds(g * _CHUNK, _CHUNK)]
            return pltpu.make_async_copy(
                table_hbm.at[idx_slice], bufs[b], gsems[b])

        def piece_copies(g, b):
            for c2 in range(npc):
                yield pltpu.make_async_copy(
                    bufs[b].at[:, pl.ds(c2 * 128, 128)],
                    out_hbm.at[pl.ds(c2 * nt + base + g * _CHUNK, _CHUNK)],
                    osems[b])

        def outcopy_start(g, b):
            for cp in piece_copies(g, b):
                cp.start()

        def outcopy_wait(g, b):
            for cp in piece_copies(g, b):
                cp.wait()

        gather(0, 0).start()
        gather(1, 1).start()

        def pair_body(p, carry):
            for b in range(2):
                g = 2 * p + b
                gather(g, b).wait()
                outcopy_start(g, b)

                @pl.when(g + 2 < n_chunks)
                def _():
                    outcopy_wait(g, b)
                    gather(g + 2, b).start()
            return carry

        lax.fori_loop(0, n_chunks // 2, pair_body, 0)
        outcopy_wait(n_chunks - 2, 0)
        outcopy_wait(n_chunks - 1, 1)

        for eg in elem_gathers():
            eg.wait()
        acc_v[...] = jnp.zeros((_LANES,), jnp.float32)

        def loss_body(i, carry):
            o = i * _LANES
            acc_v[...] = acc_v[...] + (
                lsetok_v[pl.ds(o, _LANES)] - picked_v[pl.ds(o, _LANES)])
            return carry

        lax.fori_loop(0, per_w // _LANES, loss_body, 0)
        pltpu.sync_copy(acc_v, part_hbm.at[pl.ds(wid * _LANES, _LANES)])

    return sc_kernel


def kernel(idx, targets, table):
    b, t = idx.shape
    v, d = table.shape
    nt = b * t
    dp = 1024
    lse = _row_lse(table).reshape(v)
    idx_f = jnp.transpose(idx).reshape(nt).astype(jnp.int32)
    fidx = v + idx_f * d + jnp.transpose(targets).reshape(nt).astype(
        jnp.int32)
    cat = jnp.concatenate([lse, table.reshape(v * d)])
    table_p = jnp.pad(table, ((0, 0), (0, dp - d)))
    info = plsc.get_sparse_core_info()
    sck = _make_sc_kernel(nt, v, d, dp, info.num_cores, info.num_subcores)
    pieces, parts = sck(table_p, cat, idx_f, fidx)
    loss = jnp.sum(parts) / nt
    x4 = pieces.reshape(dp // 128, t, b, 128)
    out_t = _retile(x4, v)
    return jnp.transpose(out_t, (2, 0, 1)), loss


# speedup vs baseline: 6.0973x; 1.5311x over previous
import functools

import jax
import jax.numpy as jnp
from jax import lax
from jax.experimental import pallas as pl
from jax.experimental.pallas import tpu as pltpu
from jax.experimental.pallas import tpu_sc as plsc

_LANES = 16
_CHUNK = 32


def _retile_body(in_ref, out_ref):
    npc = in_ref.shape[0]
    v = out_ref.shape[1]
    for c2 in range(npc):
        x = in_ref[c2, 0, :, :]
        xt = jnp.transpose(x)
        n = min(128, v - c2 * 128)
        out_ref[0, c2 * 128:c2 * 128 + n, :] = xt[:n, :]


def _retile(x4, v):
    npc, tt, bb = x4.shape[0], x4.shape[1], x4.shape[2]
    return pl.pallas_call(
        _retile_body,
        grid=(tt,),
        in_specs=[pl.BlockSpec((npc, 1, bb, 128), lambda t: (0, t, 0, 0))],
        out_specs=pl.BlockSpec((1, v, bb), lambda t: (t, 0, 0)),
        out_shape=jax.ShapeDtypeStruct((tt, v, bb), jnp.float32),
    )(x4)


def _lse_body(table_ref, lse_ref):
    t = table_ref[...]
    m = jnp.max(t, axis=1, keepdims=True)
    s = jnp.sum(jnp.exp(t - m), axis=1, keepdims=True)
    lse_ref[...] = jnp.log(s) + m


def _row_lse(table):
    v = table.shape[0]
    return pl.pallas_call(
        _lse_body,
        out_shape=jax.ShapeDtypeStruct((v, 1), jnp.float32),
    )(table)


@functools.lru_cache(maxsize=None)
def _make_sc_kernel(nt, v, d, dp, nc, ns):
    nw = nc * ns
    per_w = nt // nw
    assert per_w * nw == nt
    n_chunks = per_w // _CHUNK
    assert n_chunks * _CHUNK == per_w and n_chunks % 2 == 0
    npc = dp // 128

    mesh = plsc.VectorSubcoreMesh(core_axis_name="c", subcore_axis_name="s")

    @functools.partial(
        pl.kernel,
        mesh=mesh,
        compiler_params=pltpu.CompilerParams(use_tc_tiling_on_sc=False),
        out_type=[
            jax.ShapeDtypeStruct((npc * nt, 128), jnp.float32),
            jax.ShapeDtypeStruct((nw * _LANES,), jnp.float32),
        ],
        scratch_types=[
            pltpu.VMEM((per_w,), jnp.int32),
            pltpu.VMEM((per_w,), jnp.int32),
            pltpu.VMEM((per_w,), jnp.float32),
            pltpu.VMEM((per_w,), jnp.float32),
            pltpu.VMEM((_CHUNK, dp), jnp.float32),
            pltpu.VMEM((_CHUNK, dp), jnp.float32),
            pltpu.VMEM((_LANES,), jnp.float32),
            pltpu.SemaphoreType.DMA,
            pltpu.SemaphoreType.DMA,
            pltpu.SemaphoreType.DMA,
            pltpu.SemaphoreType.DMA,
            pltpu.SemaphoreType.DMA,
        ],
    )
    def sc_kernel(table_hbm, cat_hbm, idx_hbm, fidx_hbm,
                  out_hbm, part_hbm,
                  idx_v, fidx_v, picked_v, lsetok_v, buf0, buf1, acc_v,
                  gsem0, gsem1, osem0, osem1, psem):
        wid = lax.axis_index("s") * nc + lax.axis_index("c")
        base = wid * per_w
        pltpu.sync_copy(idx_hbm.at[pl.ds(base, per_w)], idx_v)
        pltpu.sync_copy(fidx_hbm.at[pl.ds(base, per_w)], fidx_v)

        def elem_gathers():
            for k in range(per_w // 128):
                s = pl.ds(k * 128, 128)
                yield pltpu.make_async_copy(
                    cat_hbm.at[fidx_v.at[s]], picked_v.at[s], psem)
                yield pltpu.make_async_copy(
                    cat_hbm.at[idx_v.at[s]], lsetok_v.at[s], psem)

        for eg in elem_gathers():
            eg.start()

        bufs = (buf0, buf1)
        gsems = (gsem0, gsem1)
        osems = (osem0, osem1)

        def gather(g, b):
            idx_slice = idx_v.at[pl.ds(g * _CHUNK, _CHUNK)]
            return pltpu.make_async_copy(
                table_hbm.at[idx_slice], bufs[b], gsems[b])

        def piece_copies(g, b):
            for c2 in range(npc):
                yield pltpu.make_async_copy(
                    bufs[b].at[:, pl.ds(c2 * 128, 128)],
                    out_hbm.at[pl.ds(c2 * nt + base + g * _CHUNK, _CHUNK)],
                    osems[b])

        def outcopy_start(g, b):
            for cp in piece_copies(g, b):
                cp.start()

        def outcopy_wait(g, b):
            for cp in piece_copies(g, b):
                cp.wait()

        gather(0, 0).start()
        gather(1, 1).start()

        def pair_body(p, carry):
            for b in range(2):
                g = 2 * p + b
                gather(g, b).wait()
                outcopy_start(g, b)

                @pl.when(g + 2 < n_chunks)
                def _():
                    outcopy_wait(g, b)
                    gather(g + 2, b).start()
            return carry

        lax.fori_loop(0, n_chunks // 2, pair_body, 0)
        outcopy_wait(n_chunks - 2, 0)
        outcopy_wait(n_chunks - 1, 1)

        for eg in elem_gathers():
            eg.wait()
        acc_v[...] = jnp.zeros((_LANES,), jnp.float32)

        def loss_body(i, carry):
            o = i * _LANES
            acc_v[...] = acc_v[...] + (
                lsetok_v[pl.ds(o, _LANES)] - picked_v[pl.ds(o, _LANES)])
            return carry

        lax.fori_loop(0, per_w // _LANES, loss_body, 0)
        pltpu.sync_copy(acc_v, part_hbm.at[pl.ds(wid * _LANES, _LANES)])

    return sc_kernel


def kernel(idx, targets, table):
    b, t = idx.shape
    v, d = table.shape
    nt = b * t
    dp = 1024
    lse = _row_lse(table).reshape(v)
    idx_f = jnp.transpose(idx).reshape(nt).astype(jnp.int32)
    fidx = v + idx_f * d + jnp.transpose(targets).reshape(nt).astype(
        jnp.int32)
    cat = jnp.concatenate([lse, table.reshape(v * d)])
    table_p = jnp.pad(table, ((0, 0), (0, dp - d)))
    info = plsc.get_sparse_core_info()
    sck = _make_sc_kernel(nt, v, d, dp, info.num_cores, info.num_subcores)
    pieces, parts = sck(table_p, cat, idx_f, fidx)
    loss = jnp.sum(parts) / nt
    x4 = pieces.reshape(dp // 128, t, b, 128)
    out_t = _retile(x4, v)
    return jnp.transpose(out_t, (2, 0, 1)), loss


# speedup vs baseline: 6.1050x; 1.0013x over previous
import functools

import jax
import jax.numpy as jnp
from jax import lax
from jax.experimental import pallas as pl
from jax.experimental.pallas import tpu as pltpu
from jax.experimental.pallas import tpu_sc as plsc

_LANES = 16
_CHUNK = 32


def _retile_body(in_ref, out_ref):
    npc = in_ref.shape[0]
    v = out_ref.shape[1]
    for c2 in range(npc):
        x = in_ref[c2, 0, :, :]
        xt = jnp.transpose(x)
        n = min(128, v - c2 * 128)
        out_ref[0, c2 * 128:c2 * 128 + n, :] = xt[:n, :]


def _retile(x4, v, tt_full, t0, prev=None):
    npc, ttq, bb = x4.shape[0], x4.shape[1], x4.shape[2]
    out_shape = jax.ShapeDtypeStruct((tt_full, v, bb), jnp.float32)
    x4_spec = pl.BlockSpec((npc, 1, bb, 128), lambda tq: (0, tq, 0, 0))
    out_spec = pl.BlockSpec((1, v, bb), lambda tq: (t0 + tq, 0, 0))
    if prev is None:
        return pl.pallas_call(
            _retile_body,
            grid=(ttq,),
            in_specs=[x4_spec],
            out_specs=out_spec,
            out_shape=out_shape,
        )(x4)

    def body(_, in_ref, out_ref):
        _retile_body(in_ref, out_ref)

    return pl.pallas_call(
        body,
        grid=(ttq,),
        in_specs=[pl.BlockSpec(memory_space=pl.ANY), x4_spec],
        out_specs=out_spec,
        out_shape=out_shape,
        input_output_aliases={0: 0},
    )(prev, x4)


def _lse_body(table_ref, lse_ref):
    t = table_ref[...]
    m = jnp.max(t, axis=1, keepdims=True)
    s = jnp.sum(jnp.exp(t - m), axis=1, keepdims=True)
    lse_ref[...] = jnp.log(s) + m


def _row_lse(table):
    v = table.shape[0]
    return pl.pallas_call(
        _lse_body,
        out_shape=jax.ShapeDtypeStruct((v, 1), jnp.float32),
    )(table)


@functools.lru_cache(maxsize=None)
def _make_sc_kernel(nt, v, d, dp, nc, ns):
    nw = nc * ns
    per_w = nt // nw
    assert per_w * nw == nt
    n_chunks = per_w // _CHUNK
    assert n_chunks * _CHUNK == per_w and n_chunks % 2 == 0
    npc = dp // 128

    mesh = plsc.VectorSubcoreMesh(core_axis_name="c", subcore_axis_name="s")

    @functools.partial(
        pl.kernel,
        mesh=mesh,
        compiler_params=pltpu.CompilerParams(use_tc_tiling_on_sc=False),
        out_type=[
            jax.ShapeDtypeStruct((npc * nt, 128), jnp.float32),
            jax.ShapeDtypeStruct((nw * _LANES,), jnp.float32),
        ],
        scratch_types=[
            pltpu.VMEM((per_w,), jnp.int32),
            pltpu.VMEM((per_w,), jnp.int32),
            pltpu.VMEM((per_w,), jnp.float32),
            pltpu.VMEM((per_w,), jnp.float32),
            pltpu.VMEM((_CHUNK, dp), jnp.float32),
            pltpu.VMEM((_CHUNK, dp), jnp.float32),
            pltpu.VMEM((_LANES,), jnp.float32),
            pltpu.SemaphoreType.DMA,
            pltpu.SemaphoreType.DMA,
            pltpu.SemaphoreType.DMA,
            pltpu.SemaphoreType.DMA,
            pltpu.SemaphoreType.DMA,
        ],
    )
    def sc_kernel(table_hbm, cat_hbm, idx_hbm, fidx_hbm,
                  out_hbm, part_hbm,
                  idx_v, fidx_v, picked_v, lsetok_v, buf0, buf1, acc_v,
                  gsem0, gsem1, osem0, osem1, psem):
        wid = lax.axis_index("s") * nc + lax.axis_index("c")
        base = wid * per_w
        pltpu.sync_copy(idx_hbm.at[pl.ds(base, per_w)], idx_v)
        pltpu.sync_copy(fidx_hbm.at[pl.ds(base, per_w)], fidx_v)

        def elem_gathers():
            for k in range(per_w // 128):
                s = pl.ds(k * 128, 128)
                yield pltpu.make_async_copy(
                    cat_hbm.at[fidx_v.at[s]], picked_v.at[s], psem)
                yield pltpu.make_async_copy(
                    cat_hbm.at[idx_v.at[s]], lsetok_v.at[s], psem)

        for eg in elem_gathers():
            eg.start()

        bufs = (buf0, buf1)
        gsems = (gsem0, gsem1)
        osems = (osem0, osem1)

        def gather(g, b):
            idx_slice = idx_v.at[pl.ds(g * _CHUNK, _CHUNK)]
            return pltpu.make_async_copy(
                table_hbm.at[idx_slice], bufs[b], gsems[b])

        def piece_copies(g, b):
            for c2 in range(npc):
                yield pltpu.make_async_copy(
                    bufs[b].at[:, pl.ds(c2 * 128, 128)],
                    out_hbm.at[pl.ds(c2 * nt + base + g * _CHUNK, _CHUNK)],
                    osems[b])

        def outcopy_start(g, b):
            for cp in piece_copies(g, b):
                cp.start()

        def outcopy_wait(g, b):
            for cp in piece_copies(g, b):
                cp.wait()

        gather(0, 0).start()
        gather(1, 1).start()

        def pair_body(p, carry):
            for b in range(2):
                g = 2 * p + b
                gather(g, b).wait()
                outcopy_start(g, b)

                @pl.when(g + 2 < n_chunks)
                def _():
                    outcopy_wait(g, b)
                    gather(g + 2, b).start()
            return carry

        lax.fori_loop(0, n_chunks // 2, pair_body, 0)
        outcopy_wait(n_chunks - 2, 0)
        outcopy_wait(n_chunks - 1, 1)

        for eg in elem_gathers():
            eg.wait()
        acc_v[...] = jnp.zeros((_LANES,), jnp.float32)

        def loss_body(i, carry):
            o = i * _LANES
            acc_v[...] = acc_v[...] + (
                lsetok_v[pl.ds(o, _LANES)] - picked_v[pl.ds(o, _LANES)])
            return carry

        lax.fori_loop(0, per_w // _LANES, loss_body, 0)
        pltpu.sync_copy(acc_v, part_hbm.at[pl.ds(wid * _LANES, _LANES)])

    return sc_kernel


def kernel(idx, targets, table):
    b, t = idx.shape
    v, d = table.shape
    nt = b * t
    dp = 1024
    lse = _row_lse(table).reshape(v)
    idx_f = jnp.transpose(idx).reshape(nt).astype(jnp.int32)
    fidx = v + idx_f * d + jnp.transpose(targets).reshape(nt).astype(
        jnp.int32)
    cat = jnp.concatenate([lse, table.reshape(v * d)])
    table_p = jnp.pad(table, ((0, 0), (0, dp - d)))
    info = plsc.get_sparse_core_info()
    nq = 2
    ntq = nt // nq
    ttq = t // nq
    sck = _make_sc_kernel(ntq, v, d, dp, info.num_cores, info.num_subcores)
    npc = dp // 128
    out_t = None
    loss_sum = 0.0
    for q in range(nq):
        pieces, parts = sck(table_p, cat, idx_f[q * ntq:(q + 1) * ntq],
                            fidx[q * ntq:(q + 1) * ntq])
        loss_sum = loss_sum + jnp.sum(parts)
        x4 = pieces.reshape(npc, ttq, b, 128)
        out_t = _retile(x4, v, t, q * ttq, prev=out_t)
    loss = loss_sum / nt
    return jnp.transpose(out_t, (2, 0, 1)), loss


# speedup vs baseline: 6.1065x; 1.0002x over previous
import functools

import jax
import jax.numpy as jnp
from jax import lax
from jax.experimental import pallas as pl
from jax.experimental.pallas import tpu as pltpu
from jax.experimental.pallas import tpu_sc as plsc

_LANES = 16
_CHUNK = 32


def _retile_body(in_ref, out_ref):
    npc = in_ref.shape[0]
    v = out_ref.shape[1]
    for c2 in range(npc):
        x = in_ref[c2, 0, :, :]
        xt = jnp.transpose(x)
        n = min(128, v - c2 * 128)
        out_ref[0, c2 * 128:c2 * 128 + n, :] = xt[:n, :]


def _retile(x4, v, tt_full, t0, prev=None):
    npc, ttq, bb = x4.shape[0], x4.shape[1], x4.shape[2]
    out_shape = jax.ShapeDtypeStruct((tt_full, v, bb), jnp.float32)
    x4_spec = pl.BlockSpec((npc, 1, bb, 128), lambda tq: (0, tq, 0, 0))
    out_spec = pl.BlockSpec((1, v, bb), lambda tq: (t0 + tq, 0, 0))
    cost = pl.CostEstimate(
        flops=0, transcendentals=0,
        bytes_accessed=2 * npc * ttq * bb * 128 * 4)
    if prev is None:
        return pl.pallas_call(
            _retile_body,
            grid=(ttq,),
            in_specs=[x4_spec],
            out_specs=out_spec,
            out_shape=out_shape,
            cost_estimate=cost,
        )(x4)

    def body(_, in_ref, out_ref):
        _retile_body(in_ref, out_ref)

    return pl.pallas_call(
        body,
        grid=(ttq,),
        in_specs=[pl.BlockSpec(memory_space=pl.ANY), x4_spec],
        out_specs=out_spec,
        out_shape=out_shape,
        input_output_aliases={0: 0},
        cost_estimate=cost,
    )(prev, x4)


def _lse_body(table_ref, lse_ref):
    t = table_ref[...]
    m = jnp.max(t, axis=1, keepdims=True)
    s = jnp.sum(jnp.exp(t - m), axis=1, keepdims=True)
    lse_ref[...] = jnp.log(s) + m


def _row_lse(table):
    v = table.shape[0]
    return pl.pallas_call(
        _lse_body,
        out_shape=jax.ShapeDtypeStruct((v, 1), jnp.float32),
    )(table)


@functools.lru_cache(maxsize=None)
def _make_sc_kernel(nt, v, d, dp, nc, ns):
    nw = nc * ns
    per_w = nt // nw
    assert per_w * nw == nt
    n_chunks = per_w // _CHUNK
    assert n_chunks * _CHUNK == per_w and n_chunks % 2 == 0
    npc = dp // 128

    mesh = plsc.VectorSubcoreMesh(core_axis_name="c", subcore_axis_name="s")

    @functools.partial(
        pl.kernel,
        mesh=mesh,
        compiler_params=pltpu.CompilerParams(use_tc_tiling_on_sc=False),
        out_type=[
            jax.ShapeDtypeStruct((npc * nt, 128), jnp.float32),
            jax.ShapeDtypeStruct((nw * _LANES,), jnp.float32),
        ],
        scratch_types=[
            pltpu.VMEM((per_w,), jnp.int32),
            pltpu.VMEM((per_w,), jnp.int32),
            pltpu.VMEM((per_w,), jnp.float32),
            pltpu.VMEM((per_w,), jnp.float32),
            pltpu.VMEM((_CHUNK, dp), jnp.float32),
            pltpu.VMEM((_CHUNK, dp), jnp.float32),
            pltpu.VMEM((_LANES,), jnp.float32),
            pltpu.SemaphoreType.DMA,
            pltpu.SemaphoreType.DMA,
            pltpu.SemaphoreType.DMA,
            pltpu.SemaphoreType.DMA,
            pltpu.SemaphoreType.DMA,
        ],
    )
    def sc_kernel(table_hbm, cat_hbm, idx_hbm, fidx_hbm,
                  out_hbm, part_hbm,
                  idx_v, fidx_v, picked_v, lsetok_v, buf0, buf1, acc_v,
                  gsem0, gsem1, osem0, osem1, psem):
        wid = lax.axis_index("s") * nc + lax.axis_index("c")
        base = wid * per_w
        pltpu.sync_copy(idx_hbm.at[pl.ds(base, per_w)], idx_v)
        pltpu.sync_copy(fidx_hbm.at[pl.ds(base, per_w)], fidx_v)

        def elem_gathers():
            for k in range(per_w // 128):
                s = pl.ds(k * 128, 128)
                yield pltpu.make_async_copy(
                    cat_hbm.at[fidx_v.at[s]], picked_v.at[s], psem)
                yield pltpu.make_async_copy(
                    cat_hbm.at[idx_v.at[s]], lsetok_v.at[s], psem)

        for eg in elem_gathers():
            eg.start()

        bufs = (buf0, buf1)
        gsems = (gsem0, gsem1)
        osems = (osem0, osem1)

        def gather(g, b):
            idx_slice = idx_v.at[pl.ds(g * _CHUNK, _CHUNK)]
            return pltpu.make_async_copy(
                table_hbm.at[idx_slice], bufs[b], gsems[b])

        def piece_copies(g, b):
            for c2 in range(npc):
                yield pltpu.make_async_copy(
                    bufs[b].at[:, pl.ds(c2 * 128, 128)],
                    out_hbm.at[pl.ds(c2 * nt + base + g * _CHUNK, _CHUNK)],
                    osems[b])

        def outcopy_start(g, b):
            for cp in piece_copies(g, b):
                cp.start()

        def outcopy_wait(g, b):
            for cp in piece_copies(g, b):
                cp.wait()

        gather(0, 0).start()
        gather(1, 1).start()

        def pair_body(p, carry):
            for b in range(2):
                g = 2 * p + b
                gather(g, b).wait()
                outcopy_start(g, b)

                @pl.when(g + 2 < n_chunks)
                def _():
                    outcopy_wait(g, b)
                    gather(g + 2, b).start()
            return carry

        lax.fori_loop(0, n_chunks // 2, pair_body, 0)
        outcopy_wait(n_chunks - 2, 0)
        outcopy_wait(n_chunks - 1, 1)

        for eg in elem_gathers():
            eg.wait()
        acc_v[...] = jnp.zeros((_LANES,), jnp.float32)

        def loss_body(i, carry):
            o = i * _LANES
            acc_v[...] = acc_v[...] + (
                lsetok_v[pl.ds(o, _LANES)] - picked_v[pl.ds(o, _LANES)])
            return carry

        lax.fori_loop(0, per_w // _LANES, loss_body, 0)
        pltpu.sync_copy(acc_v, part_hbm.at[pl.ds(wid * _LANES, _LANES)])

    return sc_kernel


def kernel(idx, targets, table):
    b, t = idx.shape
    v, d = table.shape
    nt = b * t
    dp = 1024
    lse = _row_lse(table).reshape(v)
    idx_f = jnp.transpose(idx).reshape(nt).astype(jnp.int32)
    fidx = v + idx_f * d + jnp.transpose(targets).reshape(nt).astype(
        jnp.int32)
    cat = jnp.concatenate([lse, table.reshape(v * d)])
    table_p = jnp.pad(table, ((0, 0), (0, dp - d)))
    info = plsc.get_sparse_core_info()
    nq = 2
    ntq = nt // nq
    ttq = t // nq
    sck = _make_sc_kernel(ntq, v, d, dp, info.num_cores, info.num_subcores)
    npc = dp // 128
    out_t = None
    loss_sum = 0.0
    for q in range(nq):
        pieces, parts = sck(table_p, cat, idx_f[q * ntq:(q + 1) * ntq],
                            fidx[q * ntq:(q + 1) * ntq])
        loss_sum = loss_sum + jnp.sum(parts)
        x4 = pieces.reshape(npc, ttq, b, 128)
        out_t = _retile(x4, v, t, q * ttq, prev=out_t)
    loss = loss_sum / nt
    return jnp.transpose(out_t, (2, 0, 1)), loss
